# baseline (speedup 1.0000x reference)
import functools

import jax
import jax.numpy as jnp
from jax import lax
from jax.experimental import pallas as pl
from jax.experimental.pallas import tpu as pltpu
from jax.experimental.pallas import tpu_sc as plsc

_CH = 128
_L = 16


@functools.cache
def _build(B, E, D):
    info = plsc.get_sparse_core_info()
    nw = info.num_cores * info.num_subcores
    nc = info.num_cores
    bpw = B // nw
    n_chunks = bpw // _CH
    mesh = plsc.VectorSubcoreMesh(core_axis_name="c", subcore_axis_name="s")

    @functools.partial(
        pl.kernel,
        mesh=mesh,
        out_type=jax.ShapeDtypeStruct((B, D), jnp.float32),
        compiler_params=pltpu.CompilerParams(
            needs_layout_passes=False, use_tc_tiling_on_sc=False),
        scratch_types=[
            pltpu.VMEM((n_chunks, _CH), jnp.int32),
            pltpu.VMEM((n_chunks, _CH), jnp.int32),
            pltpu.VMEM((n_chunks, _CH), jnp.int32),
            pltpu.VMEM((bpw,), jnp.float32),
            pltpu.VMEM((_CH, E), jnp.float32),
            pltpu.VMEM((_CH, E), jnp.float32),
            pltpu.VMEM((_CH, E), jnp.float32),
            pltpu.VMEM((_CH, D), jnp.float32),
            pltpu.SemaphoreType.DMA,
        ],
    )
    def k(a_idx_hbm, p_idx_hbm, s_idx_hbm, dang_hbm, a_tab, p_tab, s_tab,
          out_hbm, a_idx, p_idx, s_idx, d_v, a_rows, p_rows, s_rows, out_c,
          sem):
        wid = lax.axis_index("s") * nc + lax.axis_index("c")
        base = wid * bpw
        cbase = wid * n_chunks
        pltpu.sync_copy(a_idx_hbm.at[pl.ds(cbase, n_chunks)], a_idx)
        pltpu.sync_copy(p_idx_hbm.at[pl.ds(cbase, n_chunks)], p_idx)
        pltpu.sync_copy(s_idx_hbm.at[pl.ds(cbase, n_chunks)], s_idx)
        pltpu.sync_copy(dang_hbm.at[pl.ds(base, bpw)], d_v)
        lanes = lax.iota(jnp.int32, _L)
        dcol = jnp.full((_L,), 3 * E, jnp.int32)
        for j in range(n_chunks):
            gathers = [
                pltpu.async_copy(a_tab.at[a_idx.at[j]], a_rows, sem),
                pltpu.async_copy(p_tab.at[p_idx.at[j]], p_rows, sem),
                pltpu.async_copy(s_tab.at[s_idx.at[j]], s_rows, sem),
            ]
            for c in gathers:
                c.wait()

            @plsc.parallel_loop(0, _CH, unroll=4)
            def _(r):
                for t, buf in enumerate((a_rows, p_rows, s_rows)):
                    for h in range(E // _L):
                        out_c[r, pl.ds(t * E + h * _L, _L)] = (
                            buf[r, pl.ds(h * _L, _L)])

            for kk in range(_CH // _L):
                d16 = d_v[pl.ds(j * _CH + kk * _L, _L)]
                plsc.store_scatter(out_c, [lanes + kk * _L, dcol], d16)

            pltpu.sync_copy(out_c, out_hbm.at[pl.ds(base + j * _CH, _CH)])

    return k


def kernel(obs, action_embeddings, parent_embeddings, sibling_embeddings):
    B = obs.shape[0]
    E = action_embeddings.shape[1]
    a = obs[:, 0].astype(jnp.int32).reshape(B // _CH, _CH)
    p = obs[:, 1].astype(jnp.int32).reshape(B // _CH, _CH)
    s = obs[:, 2].astype(jnp.int32).reshape(B // _CH, _CH)
    d = obs[:, 3]
    return _build(B, E, 3 * E + 1)(
        a, p, s, d, action_embeddings, parent_embeddings, sibling_embeddings)
